# Initial kernel scaffold; baseline (speedup 1.0000x reference)
import functools

import jax
import jax.numpy as jnp
from jax import lax
from jax.experimental import pallas as pl
from jax.experimental.pallas import tpu as pltpu
from jax.experimental.pallas import tpu_sc as plsc

NC = 2
NS = 16
NW = NC * NS


def _pick_chunk(ept: int) -> int:
    best = 0
    for ch in range(8, 8193, 8):
        if ept % ch == 0:
            best = ch
    return best


def _sc_mesh():
    return plsc.VectorSubcoreMesh(core_axis_name="c", subcore_axis_name="s")


def _make_degree_kernel(E, NP, EPT, CH):
    nchunk = EPT // CH
    rpt = NP // NS

    @functools.partial(
        pl.kernel,
        out_type=jax.ShapeDtypeStruct((NC, NP), jnp.float32),
        mesh=_sc_mesh(),
        scratch_types=[
            pltpu.VMEM((CH,), jnp.int32),
            pltpu.VMEM((CH,), jnp.float32),
            pltpu.VMEM_SHARED((NP,), jnp.float32),
            pltpu.SemaphoreType.DMA,
        ],
    )
    def deg_kernel(edge_hbm, zeros_hbm, out_hbm, idx_v, ones_v, acc_sh, sem):
        cid = lax.axis_index("c")
        sid = lax.axis_index("s")
        sl = pl.ds(sid * rpt, rpt)
        pltpu.sync_copy(zeros_hbm.at[sl], acc_sh.at[sl])
        def fill(i, _):
            ones_v[pl.ds(i * 16, 16)] = jnp.ones((16,), jnp.float32)
            return 0
        lax.fori_loop(0, CH // 16, fill, 0)
        plsc.subcore_barrier()
        base = (cid * NS + sid) * EPT
        def body(i, _):
            off = base + i * CH
            pltpu.sync_copy(edge_hbm.at[1, pl.ds(off, CH)], idx_v)
            pltpu.sync_copy(ones_v, acc_sh.at[idx_v], add=True)
            return 0
        lax.fori_loop(0, nchunk, body, 0)
        plsc.subcore_barrier()
        pltpu.sync_copy(acc_sh.at[sl], out_hbm.at[cid, sl])

    return deg_kernel


def _make_prop_kernel(E, NP, EPT, CH):
    nchunk = EPT // CH
    rpt = NP // NS

    @functools.partial(
        pl.kernel,
        out_type=jax.ShapeDtypeStruct((NC, NP, 2), jnp.float32),
        mesh=_sc_mesh(),
        scratch_types=[
            pltpu.VMEM((CH,), jnp.int32),
            pltpu.VMEM((CH,), jnp.int32),
            pltpu.VMEM((CH, 2), jnp.float32),
            pltpu.VMEM_SHARED((NP, 2), jnp.float32),
            pltpu.VMEM_SHARED((NP, 2), jnp.float32),
            pltpu.SemaphoreType.DMA,
        ],
    )
    def prop_kernel(edge_hbm, g_hbm, zeros_hbm, out_hbm,
                    si_v, di_v, rows_v, tab_sh, acc_sh, sem):
        cid = lax.axis_index("c")
        sid = lax.axis_index("s")
        sl = pl.ds(sid * rpt, rpt)
        pltpu.sync_copy(g_hbm.at[sl], tab_sh.at[sl])
        pltpu.sync_copy(zeros_hbm.at[sl], acc_sh.at[sl])
        plsc.subcore_barrier()
        base = (cid * NS + sid) * EPT
        def body(i, _):
            off = base + i * CH
            pltpu.sync_copy(edge_hbm.at[0, pl.ds(off, CH)], si_v)
            pltpu.sync_copy(edge_hbm.at[1, pl.ds(off, CH)], di_v)
            pltpu.async_copy(tab_sh.at[si_v], rows_v, sem).wait()
            pltpu.sync_copy(rows_v, acc_sh.at[di_v], add=True)
            return 0
        lax.fori_loop(0, nchunk, body, 0)
        plsc.subcore_barrier()
        pltpu.sync_copy(acc_sh.at[sl], out_hbm.at[cid, sl])

    return prop_kernel


def _dense1_body(degp_ref, xp_ref, dis_ref, g1_ref):
    deg = degp_ref[0:1, :] + degp_ref[1:2, :] + 1.0
    dis = lax.rsqrt(deg)
    dis_t = jnp.transpose(dis)
    dis_ref[...] = dis_t
    g1_ref[...] = xp_ref[...] * dis_t


def _dense2_body(p_ref, g1_ref, dis_ref, W1_ref, b1_ref, W2_ref, b2_ref, g2_ref):
    dis = dis_ref[...]
    s1 = dis * (p_ref[0] + p_ref[1] + g1_ref[...])
    W1 = W1_ref[...]
    h = (s1[:, 0:1] * W1[0:1, :] + s1[:, 1:2] * W1[1:2, :]
         + b1_ref[...][None, :])
    h = jnp.maximum(h, 0.0)
    W2 = W2_ref[...]
    t = (h[:, 0:1] * W2[0:1, :] + h[:, 1:2] * W2[1:2, :]
         + h[:, 2:3] * W2[2:3, :] + h[:, 3:4] * W2[3:4, :])
    g2_ref[...] = t * dis


def _make_dense3_body(n):
    def _dense3_body(p_ref, g2_ref, dis_ref, b2_ref, out_ref):
        s2 = (dis_ref[...] * (p_ref[0] + p_ref[1] + g2_ref[...])
              + b2_ref[...][None, :])
        m = jnp.max(s2, axis=1, keepdims=True)
        z = s2 - m
        lse = jnp.log(jnp.exp(z[:, 0:1]) + jnp.exp(z[:, 1:2]))
        out_ref[...] = (z - lse)[:n, :]
    return _dense3_body


def kernel(x, edge_index, W1, b1, W2, b2):
    n, f_in = x.shape
    e = edge_index.shape[1]
    assert e % NW == 0, "edge count must split across the 32 subcores"
    ept = e // NW
    ch = _pick_chunk(ept)
    assert ch > 0, "no legal chunk size"
    npad = ((n + NS * 8 - 1) // (NS * 8)) * (NS * 8)

    xp = jnp.concatenate(
        [x, jnp.zeros((npad - n, f_in), jnp.float32)], axis=0)
    zeros1 = jnp.zeros((npad,), jnp.float32)
    zeros2 = jnp.zeros((npad, 2), jnp.float32)

    deg_k = _make_degree_kernel(e, npad, ept, ch)
    prop_k = _make_prop_kernel(e, npad, ept, ch)

    degp = deg_k(edge_index, zeros1)

    dis, g1 = pl.pallas_call(
        _dense1_body,
        out_shape=[
            jax.ShapeDtypeStruct((npad, 1), jnp.float32),
            jax.ShapeDtypeStruct((npad, 2), jnp.float32),
        ],
    )(degp, xp)

    p1 = prop_k(edge_index, g1, zeros2)

    g2 = pl.pallas_call(
        _dense2_body,
        out_shape=jax.ShapeDtypeStruct((npad, 2), jnp.float32),
    )(p1, g1, dis, W1, b1, W2, b2)

    p2 = prop_k(edge_index, g2, zeros2)

    out = pl.pallas_call(
        _make_dense3_body(n),
        out_shape=jax.ShapeDtypeStruct((n, 2), jnp.float32),
    )(p2, g2, dis, b2)
    return out


# speedup vs baseline: 41.1062x; 41.1062x over previous
import functools

import jax
import jax.numpy as jnp
from jax import lax
from jax.experimental import pallas as pl
from jax.experimental.pallas import tpu as pltpu
from jax.experimental.pallas import tpu_sc as plsc

NC = 2
NS = 16
NW = NC * NS


GRP = 128
KCH = 8


def _sc_mesh():
    return plsc.VectorSubcoreMesh(core_axis_name="c", subcore_axis_name="s")


_SC_PARAMS = pltpu.CompilerParams(use_tc_tiling_on_sc=False)


def _make_degree_kernel(NP, RPW):
    RH = RPW * NW
    nchunk = RPW // KCH
    rpt = NP // NS

    @functools.partial(
        pl.kernel,
        out_type=[jax.ShapeDtypeStruct((NP,), jnp.float32),
                  jax.ShapeDtypeStruct((NP,), jnp.float32)],
        mesh=_sc_mesh(),
        compiler_params=_SC_PARAMS,
        scratch_types=[
            pltpu.VMEM((KCH, GRP), jnp.int32),
            pltpu.VMEM((GRP,), jnp.float32),
            pltpu.VMEM((rpt,), jnp.float32),
            pltpu.VMEM_SHARED((NP,), jnp.float32),
            pltpu.SemaphoreType.DMA,
        ],
    )
    def deg_kernel(edge_hbm, zeros_hbm, out0_hbm, out1_hbm,
                   idx_v, ones_v, stage_v, acc_sh, sem):
        cid = lax.axis_index("c")
        sid = lax.axis_index("s")
        sl = pl.ds(sid * rpt, rpt)
        pltpu.sync_copy(zeros_hbm.at[sl], stage_v)
        pltpu.sync_copy(stage_v, acc_sh.at[sl])
        def fill(i, _):
            ones_v[pl.ds(i * 16, 16)] = jnp.ones((16,), jnp.float32)
            return 0
        lax.fori_loop(0, GRP // 16, fill, 0)
        plsc.subcore_barrier()
        base = RH + (cid * NS + sid) * RPW
        def body(i, _):
            pltpu.sync_copy(edge_hbm.at[pl.ds(base + i * KCH, KCH), :], idx_v)
            for j in range(KCH):
                pltpu.sync_copy(ones_v, acc_sh.at[idx_v.at[j]], add=True)
            return 0
        lax.fori_loop(0, nchunk, body, 0)
        plsc.subcore_barrier()
        pltpu.sync_copy(acc_sh.at[sl], stage_v)
        @pl.when(cid == 0)
        def _():
            pltpu.sync_copy(stage_v, out0_hbm.at[sl])
        @pl.when(cid == 1)
        def _():
            pltpu.sync_copy(stage_v, out1_hbm.at[sl])

    return deg_kernel


def _make_prop_kernel(NP, RPW):
    RH = RPW * NW
    nchunk = RPW // KCH
    rpt = NP // NS

    @functools.partial(
        pl.kernel,
        out_type=[jax.ShapeDtypeStruct((NP,), jnp.float32),
                  jax.ShapeDtypeStruct((NP,), jnp.float32),
                  jax.ShapeDtypeStruct((NP,), jnp.float32),
                  jax.ShapeDtypeStruct((NP,), jnp.float32)],
        mesh=_sc_mesh(),
        compiler_params=_SC_PARAMS,
        scratch_types=[
            pltpu.VMEM((KCH, GRP), jnp.int32),
            pltpu.VMEM((KCH, GRP), jnp.int32),
            pltpu.VMEM((GRP,), jnp.float32),
            pltpu.VMEM((GRP,), jnp.float32),
            pltpu.VMEM((rpt,), jnp.float32),
            pltpu.VMEM_SHARED((NP,), jnp.float32),
            pltpu.VMEM_SHARED((NP,), jnp.float32),
            pltpu.SemaphoreType.DMA,
            pltpu.SemaphoreType.DMA,
        ],
    )
    def prop_kernel(edge_hbm, g0_hbm, g1_hbm, zeros_hbm,
                    o00_hbm, o01_hbm, o10_hbm, o11_hbm,
                    si_v, di_v, r0_v, r1_v, stage_v, acc0_sh, acc1_sh,
                    sem0, sem1):
        cid = lax.axis_index("c")
        sid = lax.axis_index("s")
        sl = pl.ds(sid * rpt, rpt)
        pltpu.sync_copy(zeros_hbm.at[sl], stage_v)
        pltpu.sync_copy(stage_v, acc0_sh.at[sl])
        pltpu.sync_copy(stage_v, acc1_sh.at[sl])
        plsc.subcore_barrier()
        wbase = (cid * NS + sid) * RPW
        def body(i, _):
            r0 = wbase + i * KCH
            pltpu.sync_copy(edge_hbm.at[pl.ds(r0, KCH), :], si_v)
            pltpu.sync_copy(edge_hbm.at[pl.ds(RH + r0, KCH), :], di_v)
            for j in range(KCH):
                ga = pltpu.async_copy(g0_hbm.at[si_v.at[j]], r0_v, sem0)
                gb = pltpu.async_copy(g1_hbm.at[si_v.at[j]], r1_v, sem1)
                ga.wait()
                gb.wait()
                pltpu.sync_copy(r0_v, acc0_sh.at[di_v.at[j]], add=True)
                pltpu.sync_copy(r1_v, acc1_sh.at[di_v.at[j]], add=True)
            return 0
        lax.fori_loop(0, nchunk, body, 0)
        plsc.subcore_barrier()
        @pl.when(cid == 0)
        def _():
            pltpu.sync_copy(acc0_sh.at[sl], stage_v)
            pltpu.sync_copy(stage_v, o00_hbm.at[sl])
            pltpu.sync_copy(acc1_sh.at[sl], stage_v)
            pltpu.sync_copy(stage_v, o01_hbm.at[sl])
        @pl.when(cid == 1)
        def _():
            pltpu.sync_copy(acc0_sh.at[sl], stage_v)
            pltpu.sync_copy(stage_v, o10_hbm.at[sl])
            pltpu.sync_copy(acc1_sh.at[sl], stage_v)
            pltpu.sync_copy(stage_v, o11_hbm.at[sl])

    return prop_kernel


def _dense1_body(d0_ref, d1_ref, xT_ref, dis_ref, g1T_ref):
    deg = d0_ref[...] + d1_ref[...] + 1.0
    dis = lax.rsqrt(deg)
    dis_ref[...] = dis
    g1T_ref[...] = xT_ref[...] * dis


def _dense2_body(q00_ref, q01_ref, q10_ref, q11_ref, g1T_ref, dis_ref,
                 W1_ref, b1_ref, W2_ref, b2_ref, g2T_ref):
    dis = dis_ref[...]
    s0 = dis * (q00_ref[...] + q10_ref[...] + g1T_ref[0:1, :])
    s1 = dis * (q01_ref[...] + q11_ref[...] + g1T_ref[1:2, :])
    hs = []
    for j in range(4):
        hj = (s0 * W1_ref[0:1, j:j + 1] + s1 * W1_ref[1:2, j:j + 1]
              + b1_ref[0:1, j:j + 1])
        hs.append(jnp.maximum(hj, 0.0))
    ts = []
    for f in range(2):
        tf = (hs[0] * W2_ref[0:1, f:f + 1] + hs[1] * W2_ref[1:2, f:f + 1]
              + hs[2] * W2_ref[2:3, f:f + 1] + hs[3] * W2_ref[3:4, f:f + 1])
        ts.append(tf)
    g2T_ref[...] = jnp.concatenate(ts, axis=0) * dis


def _dense3_body(q00_ref, q01_ref, q10_ref, q11_ref, g2T_ref, dis_ref,
                 b2_ref, outT_ref):
    dis = dis_ref[...]
    s0 = dis * (q00_ref[...] + q10_ref[...] + g2T_ref[0:1, :]) + b2_ref[0:1, 0:1]
    s1 = dis * (q01_ref[...] + q11_ref[...] + g2T_ref[1:2, :]) + b2_ref[0:1, 1:2]
    m = jnp.maximum(s0, s1)
    z0, z1 = s0 - m, s1 - m
    lse = jnp.log(jnp.exp(z0) + jnp.exp(z1))
    outT_ref[...] = jnp.concatenate([z0 - lse, z1 - lse], axis=0)


def kernel(x, edge_index, W1, b1, W2, b2):
    n, f_in = x.shape
    e = edge_index.shape[1]
    npad = ((n + 64 + NS * 8 - 1) // (NS * 8)) * (NS * 8)
    rpw = -(-e // (GRP * NW * KCH)) * KCH
    ep = rpw * NW * GRP
    pad_e = ep - e
    pad_idx = npad - 64 + (jnp.arange(pad_e, dtype=edge_index.dtype) % 64)
    srcp = jnp.concatenate([edge_index[0], pad_idx])
    dstp = jnp.concatenate([edge_index[1], pad_idx])
    edges2d = jnp.concatenate([srcp, dstp]).reshape(-1, GRP)

    xTp = jnp.concatenate(
        [x, jnp.zeros((npad - n, f_in), jnp.float32)], axis=0).T
    zeros1 = jnp.zeros((npad,), jnp.float32)

    deg_k = _make_degree_kernel(npad, rpw)
    prop_k = _make_prop_kernel(npad, rpw)

    d0, d1 = deg_k(edges2d, zeros1)

    dis, g1T = pl.pallas_call(
        _dense1_body,
        out_shape=[
            jax.ShapeDtypeStruct((1, npad), jnp.float32),
            jax.ShapeDtypeStruct((2, npad), jnp.float32),
        ],
    )(d0.reshape(1, npad), d1.reshape(1, npad), xTp)

    q00, q01, q10, q11 = prop_k(edges2d, g1T[0], g1T[1], zeros1)

    g2T = pl.pallas_call(
        _dense2_body,
        out_shape=jax.ShapeDtypeStruct((2, npad), jnp.float32),
    )(q00.reshape(1, npad), q01.reshape(1, npad), q10.reshape(1, npad),
      q11.reshape(1, npad), g1T, dis, W1, b1.reshape(1, 4), W2,
      b2.reshape(1, 2))

    r00, r01, r10, r11 = prop_k(edges2d, g2T[0], g2T[1], zeros1)

    outT = pl.pallas_call(
        _dense3_body,
        out_shape=jax.ShapeDtypeStruct((2, npad), jnp.float32),
    )(r00.reshape(1, npad), r01.reshape(1, npad), r10.reshape(1, npad),
      r11.reshape(1, npad), g2T, dis, b2.reshape(1, 2))
    return outT.T[:n, :]


# speedup vs baseline: 58.9968x; 1.4352x over previous
import functools

import jax
import jax.numpy as jnp
from jax import lax
from jax.experimental import pallas as pl
from jax.experimental.pallas import tpu as pltpu
from jax.experimental.pallas import tpu_sc as plsc

NC = 2
NS = 16
NW = NC * NS


GRP = 1024
KCH = 8


def _sc_mesh():
    return plsc.VectorSubcoreMesh(core_axis_name="c", subcore_axis_name="s")


_SC_PARAMS = pltpu.CompilerParams(use_tc_tiling_on_sc=False)


def _make_degree_kernel(NP, RPW, GRP=GRP, KCH=KCH):
    RH = RPW * NW
    nchunk = RPW // KCH
    rpt = NP // NS

    @functools.partial(
        pl.kernel,
        out_type=[jax.ShapeDtypeStruct((NP,), jnp.float32),
                  jax.ShapeDtypeStruct((NP,), jnp.float32)],
        mesh=_sc_mesh(),
        compiler_params=_SC_PARAMS,
        scratch_types=[
            pltpu.VMEM((KCH, GRP), jnp.int32),
            pltpu.VMEM((GRP,), jnp.float32),
            pltpu.VMEM((rpt,), jnp.float32),
            pltpu.VMEM_SHARED((NP,), jnp.float32),
            pltpu.SemaphoreType.DMA,
        ],
    )
    def deg_kernel(edge_hbm, zeros_hbm, out0_hbm, out1_hbm,
                   idx_v, ones_v, stage_v, acc_sh, sem):
        cid = lax.axis_index("c")
        sid = lax.axis_index("s")
        sl = pl.ds(sid * rpt, rpt)
        pltpu.sync_copy(zeros_hbm.at[sl], stage_v)
        pltpu.sync_copy(stage_v, acc_sh.at[sl])
        def fill(i, _):
            ones_v[pl.ds(i * 16, 16)] = jnp.ones((16,), jnp.float32)
            return 0
        lax.fori_loop(0, GRP // 16, fill, 0)
        plsc.subcore_barrier()
        base = RH + (cid * NS + sid) * RPW
        def body(i, _):
            pltpu.sync_copy(edge_hbm.at[pl.ds(base + i * KCH, KCH), :], idx_v)
            for j in range(KCH):
                pltpu.sync_copy(ones_v, acc_sh.at[idx_v.at[j]], add=True)
            return 0
        lax.fori_loop(0, nchunk, body, 0)
        plsc.subcore_barrier()
        pltpu.sync_copy(acc_sh.at[sl], stage_v)
        @pl.when(cid == 0)
        def _():
            pltpu.sync_copy(stage_v, out0_hbm.at[sl])
        @pl.when(cid == 1)
        def _():
            pltpu.sync_copy(stage_v, out1_hbm.at[sl])

    return deg_kernel


def _make_prop_kernel(NP, RPW, GRP=GRP, KCH=KCH):
    RH = RPW * NW
    nchunk = RPW // KCH
    rpt = NP // NS

    @functools.partial(
        pl.kernel,
        out_type=[jax.ShapeDtypeStruct((NP,), jnp.float32),
                  jax.ShapeDtypeStruct((NP,), jnp.float32),
                  jax.ShapeDtypeStruct((NP,), jnp.float32),
                  jax.ShapeDtypeStruct((NP,), jnp.float32)],
        mesh=_sc_mesh(),
        compiler_params=_SC_PARAMS,
        scratch_types=[
            pltpu.VMEM((KCH, GRP), jnp.int32),
            pltpu.VMEM((KCH, GRP), jnp.int32),
            pltpu.VMEM((GRP,), jnp.float32),
            pltpu.VMEM((GRP,), jnp.float32),
            pltpu.VMEM((rpt,), jnp.float32),
            pltpu.VMEM_SHARED((NP,), jnp.float32),
            pltpu.VMEM_SHARED((NP,), jnp.float32),
            pltpu.SemaphoreType.DMA,
            pltpu.SemaphoreType.DMA,
        ],
    )
    def prop_kernel(edge_hbm, g0_hbm, g1_hbm, zeros_hbm,
                    o00_hbm, o01_hbm, o10_hbm, o11_hbm,
                    si_v, di_v, r0_v, r1_v, stage_v, acc0_sh, acc1_sh,
                    sem0, sem1):
        cid = lax.axis_index("c")
        sid = lax.axis_index("s")
        sl = pl.ds(sid * rpt, rpt)
        pltpu.sync_copy(zeros_hbm.at[sl], stage_v)
        pltpu.sync_copy(stage_v, acc0_sh.at[sl])
        pltpu.sync_copy(stage_v, acc1_sh.at[sl])
        plsc.subcore_barrier()
        wbase = (cid * NS + sid) * RPW
        def body(i, _):
            r0 = wbase + i * KCH
            pltpu.sync_copy(edge_hbm.at[pl.ds(r0, KCH), :], si_v)
            pltpu.sync_copy(edge_hbm.at[pl.ds(RH + r0, KCH), :], di_v)
            for j in range(KCH):
                ga = pltpu.async_copy(g0_hbm.at[si_v.at[j]], r0_v, sem0)
                gb = pltpu.async_copy(g1_hbm.at[si_v.at[j]], r1_v, sem1)
                ga.wait()
                gb.wait()
                pltpu.sync_copy(r0_v, acc0_sh.at[di_v.at[j]], add=True)
                pltpu.sync_copy(r1_v, acc1_sh.at[di_v.at[j]], add=True)
            return 0
        lax.fori_loop(0, nchunk, body, 0)
        plsc.subcore_barrier()
        @pl.when(cid == 0)
        def _():
            pltpu.sync_copy(acc0_sh.at[sl], stage_v)
            pltpu.sync_copy(stage_v, o00_hbm.at[sl])
            pltpu.sync_copy(acc1_sh.at[sl], stage_v)
            pltpu.sync_copy(stage_v, o01_hbm.at[sl])
        @pl.when(cid == 1)
        def _():
            pltpu.sync_copy(acc0_sh.at[sl], stage_v)
            pltpu.sync_copy(stage_v, o10_hbm.at[sl])
            pltpu.sync_copy(acc1_sh.at[sl], stage_v)
            pltpu.sync_copy(stage_v, o11_hbm.at[sl])

    return prop_kernel


def _dense1_body(d0_ref, d1_ref, xT_ref, dis_ref, g1T_ref):
    deg = d0_ref[...] + d1_ref[...] + 1.0
    dis = lax.rsqrt(deg)
    dis_ref[...] = dis
    g1T_ref[...] = xT_ref[...] * dis


def _dense2_body(q00_ref, q01_ref, q10_ref, q11_ref, g1T_ref, dis_ref,
                 W1_ref, b1_ref, W2_ref, b2_ref, g2T_ref):
    dis = dis_ref[...]
    s0 = dis * (q00_ref[...] + q10_ref[...] + g1T_ref[0:1, :])
    s1 = dis * (q01_ref[...] + q11_ref[...] + g1T_ref[1:2, :])
    hs = []
    for j in range(4):
        hj = (s0 * W1_ref[0:1, j:j + 1] + s1 * W1_ref[1:2, j:j + 1]
              + b1_ref[0:1, j:j + 1])
        hs.append(jnp.maximum(hj, 0.0))
    ts = []
    for f in range(2):
        tf = (hs[0] * W2_ref[0:1, f:f + 1] + hs[1] * W2_ref[1:2, f:f + 1]
              + hs[2] * W2_ref[2:3, f:f + 1] + hs[3] * W2_ref[3:4, f:f + 1])
        ts.append(tf)
    g2T_ref[...] = jnp.concatenate(ts, axis=0) * dis


def _dense3_body(q00_ref, q01_ref, q10_ref, q11_ref, g2T_ref, dis_ref,
                 b2_ref, outT_ref):
    dis = dis_ref[...]
    s0 = dis * (q00_ref[...] + q10_ref[...] + g2T_ref[0:1, :]) + b2_ref[0:1, 0:1]
    s1 = dis * (q01_ref[...] + q11_ref[...] + g2T_ref[1:2, :]) + b2_ref[0:1, 1:2]
    m = jnp.maximum(s0, s1)
    z0, z1 = s0 - m, s1 - m
    lse = jnp.log(jnp.exp(z0) + jnp.exp(z1))
    outT_ref[...] = jnp.concatenate([z0 - lse, z1 - lse], axis=0)


def kernel(x, edge_index, W1, b1, W2, b2):
    n, f_in = x.shape
    e = edge_index.shape[1]
    npad = ((n + 64 + NS * 8 - 1) // (NS * 8)) * (NS * 8)
    rpw = -(-e // (GRP * NW * KCH)) * KCH
    ep = rpw * NW * GRP
    pad_e = ep - e
    pad_idx = npad - 64 + (jnp.arange(pad_e, dtype=edge_index.dtype) % 64)
    srcp = jnp.concatenate([edge_index[0], pad_idx])
    dstp = jnp.concatenate([edge_index[1], pad_idx])
    edges2d = jnp.concatenate([srcp, dstp]).reshape(-1, GRP)

    xTp = jnp.concatenate(
        [x, jnp.zeros((npad - n, f_in), jnp.float32)], axis=0).T
    zeros1 = jnp.zeros((npad,), jnp.float32)

    deg_k = _make_degree_kernel(npad, rpw)
    prop_k = _make_prop_kernel(npad, rpw)

    d0, d1 = deg_k(edges2d, zeros1)

    dis, g1T = pl.pallas_call(
        _dense1_body,
        out_shape=[
            jax.ShapeDtypeStruct((1, npad), jnp.float32),
            jax.ShapeDtypeStruct((2, npad), jnp.float32),
        ],
    )(d0.reshape(1, npad), d1.reshape(1, npad), xTp)

    q00, q01, q10, q11 = prop_k(edges2d, g1T[0], g1T[1], zeros1)

    g2T = pl.pallas_call(
        _dense2_body,
        out_shape=jax.ShapeDtypeStruct((2, npad), jnp.float32),
    )(q00.reshape(1, npad), q01.reshape(1, npad), q10.reshape(1, npad),
      q11.reshape(1, npad), g1T, dis, W1, b1.reshape(1, 4), W2,
      b2.reshape(1, 2))

    r00, r01, r10, r11 = prop_k(edges2d, g2T[0], g2T[1], zeros1)

    outT = pl.pallas_call(
        _dense3_body,
        out_shape=jax.ShapeDtypeStruct((2, npad), jnp.float32),
    )(r00.reshape(1, npad), r01.reshape(1, npad), r10.reshape(1, npad),
      r11.reshape(1, npad), g2T, dis, b2.reshape(1, 2))
    return outT.T[:n, :]


# speedup vs baseline: 60.5605x; 1.0265x over previous
import functools

import jax
import jax.numpy as jnp
from jax import lax
from jax.experimental import pallas as pl
from jax.experimental.pallas import tpu as pltpu
from jax.experimental.pallas import tpu_sc as plsc

NC = 2
NS = 16
NW = NC * NS


GRP = 1024
KCH = 8
NBUF = 4


def _sc_mesh():
    return plsc.VectorSubcoreMesh(core_axis_name="c", subcore_axis_name="s")


_SC_PARAMS = pltpu.CompilerParams(use_tc_tiling_on_sc=False)


def _make_degree_kernel(NP, RPW, GRP=GRP, KCH=KCH):
    RH = RPW * NW
    nchunk = RPW // KCH
    rpt = NP // NS

    @functools.partial(
        pl.kernel,
        out_type=[jax.ShapeDtypeStruct((NP,), jnp.float32),
                  jax.ShapeDtypeStruct((NP,), jnp.float32)],
        mesh=_sc_mesh(),
        compiler_params=_SC_PARAMS,
        scratch_types=[
            pltpu.VMEM((KCH, GRP), jnp.int32),
            pltpu.VMEM((GRP,), jnp.float32),
            pltpu.VMEM((rpt,), jnp.float32),
            pltpu.VMEM_SHARED((NP,), jnp.float32),
            pltpu.SemaphoreType.DMA,
        ],
    )
    def deg_kernel(edge_hbm, zeros_hbm, out0_hbm, out1_hbm,
                   idx_v, ones_v, stage_v, acc_sh, sem):
        cid = lax.axis_index("c")
        sid = lax.axis_index("s")
        sl = pl.ds(sid * rpt, rpt)
        pltpu.sync_copy(zeros_hbm.at[sl], stage_v)
        pltpu.sync_copy(stage_v, acc_sh.at[sl])
        def fill(i, _):
            ones_v[pl.ds(i * 16, 16)] = jnp.ones((16,), jnp.float32)
            return 0
        lax.fori_loop(0, GRP // 16, fill, 0)
        plsc.subcore_barrier()
        base = RH + (cid * NS + sid) * RPW
        def body(i, _):
            pltpu.sync_copy(edge_hbm.at[pl.ds(base + i * KCH, KCH), :], idx_v)
            ss = [pltpu.async_copy(ones_v, acc_sh.at[idx_v.at[j]], sem,
                                   add=True)
                  for j in range(KCH)]
            for s in ss:
                s.wait()
            return 0
        lax.fori_loop(0, nchunk, body, 0)
        plsc.subcore_barrier()
        pltpu.sync_copy(acc_sh.at[sl], stage_v)
        @pl.when(cid == 0)
        def _():
            pltpu.sync_copy(stage_v, out0_hbm.at[sl])
        @pl.when(cid == 1)
        def _():
            pltpu.sync_copy(stage_v, out1_hbm.at[sl])

    return deg_kernel


def _make_prop_kernel(NP, RPW, GRP=GRP, KCH=KCH):
    RH = RPW * NW
    nchunk = RPW // KCH
    rpt = NP // NS

    @functools.partial(
        pl.kernel,
        out_type=[jax.ShapeDtypeStruct((NP,), jnp.float32),
                  jax.ShapeDtypeStruct((NP,), jnp.float32),
                  jax.ShapeDtypeStruct((NP,), jnp.float32),
                  jax.ShapeDtypeStruct((NP,), jnp.float32)],
        mesh=_sc_mesh(),
        compiler_params=_SC_PARAMS,
        scratch_types=[
            pltpu.VMEM((KCH, GRP), jnp.int32),
            pltpu.VMEM((KCH, GRP), jnp.int32),
            pltpu.VMEM((NBUF, GRP), jnp.float32),
            pltpu.VMEM((NBUF, GRP), jnp.float32),
            pltpu.VMEM((rpt,), jnp.float32),
            pltpu.VMEM_SHARED((NP,), jnp.float32),
            pltpu.VMEM_SHARED((NP,), jnp.float32),
            pltpu.SemaphoreType.DMA,
            pltpu.SemaphoreType.DMA,
            pltpu.SemaphoreType.DMA,
            pltpu.SemaphoreType.DMA,
        ],
    )
    def prop_kernel(edge_hbm, g0_hbm, g1_hbm, zeros_hbm,
                    o00_hbm, o01_hbm, o10_hbm, o11_hbm,
                    si_v, di_v, r0_v, r1_v, stage_v, acc0_sh, acc1_sh,
                    sem_g0, sem_g1, sem_s0, sem_s1):
        cid = lax.axis_index("c")
        sid = lax.axis_index("s")
        sl = pl.ds(sid * rpt, rpt)
        pltpu.sync_copy(zeros_hbm.at[sl], stage_v)
        pltpu.sync_copy(stage_v, acc0_sh.at[sl])
        pltpu.sync_copy(stage_v, acc1_sh.at[sl])
        plsc.subcore_barrier()
        wbase = (cid * NS + sid) * RPW
        def body(i, _):
            r0 = wbase + i * KCH
            pltpu.sync_copy(edge_hbm.at[pl.ds(r0, KCH), :], si_v)
            pltpu.sync_copy(edge_hbm.at[pl.ds(RH + r0, KCH), :], di_v)
            ga = [None] * KCH
            gb = [None] * KCH
            sa = [None] * KCH
            sb = [None] * KCH
            def fire_scatter(j):
                b = j % NBUF
                ga[j].wait()
                gb[j].wait()
                sa[j] = pltpu.async_copy(
                    r0_v.at[b], acc0_sh.at[di_v.at[j]], sem_s0, add=True)
                sb[j] = pltpu.async_copy(
                    r1_v.at[b], acc1_sh.at[di_v.at[j]], sem_s1, add=True)
            for j in range(KCH):
                b = j % NBUF
                if j >= NBUF:
                    sa[j - NBUF].wait()
                    sb[j - NBUF].wait()
                ga[j] = pltpu.async_copy(g0_hbm.at[si_v.at[j]],
                                         r0_v.at[b], sem_g0)
                gb[j] = pltpu.async_copy(g1_hbm.at[si_v.at[j]],
                                         r1_v.at[b], sem_g1)
                if j >= 1:
                    fire_scatter(j - 1)
            fire_scatter(KCH - 1)
            for j in range(max(0, KCH - NBUF), KCH):
                sa[j].wait()
                sb[j].wait()
            return 0
        lax.fori_loop(0, nchunk, body, 0)
        plsc.subcore_barrier()
        @pl.when(cid == 0)
        def _():
            pltpu.sync_copy(acc0_sh.at[sl], stage_v)
            pltpu.sync_copy(stage_v, o00_hbm.at[sl])
            pltpu.sync_copy(acc1_sh.at[sl], stage_v)
            pltpu.sync_copy(stage_v, o01_hbm.at[sl])
        @pl.when(cid == 1)
        def _():
            pltpu.sync_copy(acc0_sh.at[sl], stage_v)
            pltpu.sync_copy(stage_v, o10_hbm.at[sl])
            pltpu.sync_copy(acc1_sh.at[sl], stage_v)
            pltpu.sync_copy(stage_v, o11_hbm.at[sl])

    return prop_kernel


def _dense1_body(d0_ref, d1_ref, xT_ref, dis_ref, g1T_ref):
    deg = d0_ref[...] + d1_ref[...] + 1.0
    dis = lax.rsqrt(deg)
    dis_ref[...] = dis
    g1T_ref[...] = xT_ref[...] * dis


def _dense2_body(q00_ref, q01_ref, q10_ref, q11_ref, g1T_ref, dis_ref,
                 W1_ref, b1_ref, W2_ref, b2_ref, g2T_ref):
    dis = dis_ref[...]
    s0 = dis * (q00_ref[...] + q10_ref[...] + g1T_ref[0:1, :])
    s1 = dis * (q01_ref[...] + q11_ref[...] + g1T_ref[1:2, :])
    hs = []
    for j in range(4):
        hj = (s0 * W1_ref[0:1, j:j + 1] + s1 * W1_ref[1:2, j:j + 1]
              + b1_ref[0:1, j:j + 1])
        hs.append(jnp.maximum(hj, 0.0))
    ts = []
    for f in range(2):
        tf = (hs[0] * W2_ref[0:1, f:f + 1] + hs[1] * W2_ref[1:2, f:f + 1]
              + hs[2] * W2_ref[2:3, f:f + 1] + hs[3] * W2_ref[3:4, f:f + 1])
        ts.append(tf)
    g2T_ref[...] = jnp.concatenate(ts, axis=0) * dis


def _dense3_body(q00_ref, q01_ref, q10_ref, q11_ref, g2T_ref, dis_ref,
                 b2_ref, outT_ref):
    dis = dis_ref[...]
    s0 = dis * (q00_ref[...] + q10_ref[...] + g2T_ref[0:1, :]) + b2_ref[0:1, 0:1]
    s1 = dis * (q01_ref[...] + q11_ref[...] + g2T_ref[1:2, :]) + b2_ref[0:1, 1:2]
    m = jnp.maximum(s0, s1)
    z0, z1 = s0 - m, s1 - m
    lse = jnp.log(jnp.exp(z0) + jnp.exp(z1))
    outT_ref[...] = jnp.concatenate([z0 - lse, z1 - lse], axis=0)


def kernel(x, edge_index, W1, b1, W2, b2):
    n, f_in = x.shape
    e = edge_index.shape[1]
    npad = ((n + 64 + NS * 8 - 1) // (NS * 8)) * (NS * 8)
    rpw = -(-e // (GRP * NW * KCH)) * KCH
    ep = rpw * NW * GRP
    pad_e = ep - e
    pad_idx = npad - 64 + (jnp.arange(pad_e, dtype=edge_index.dtype) % 64)
    srcp = jnp.concatenate([edge_index[0], pad_idx])
    dstp = jnp.concatenate([edge_index[1], pad_idx])
    edges2d = jnp.concatenate([srcp, dstp]).reshape(-1, GRP)

    xTp = jnp.concatenate(
        [x, jnp.zeros((npad - n, f_in), jnp.float32)], axis=0).T
    zeros1 = jnp.zeros((npad,), jnp.float32)

    deg_k = _make_degree_kernel(npad, rpw)
    prop_k = _make_prop_kernel(npad, rpw)

    d0, d1 = deg_k(edges2d, zeros1)

    dis, g1T = pl.pallas_call(
        _dense1_body,
        out_shape=[
            jax.ShapeDtypeStruct((1, npad), jnp.float32),
            jax.ShapeDtypeStruct((2, npad), jnp.float32),
        ],
    )(d0.reshape(1, npad), d1.reshape(1, npad), xTp)

    q00, q01, q10, q11 = prop_k(edges2d, g1T[0], g1T[1], zeros1)

    g2T = pl.pallas_call(
        _dense2_body,
        out_shape=jax.ShapeDtypeStruct((2, npad), jnp.float32),
    )(q00.reshape(1, npad), q01.reshape(1, npad), q10.reshape(1, npad),
      q11.reshape(1, npad), g1T, dis, W1, b1.reshape(1, 4), W2,
      b2.reshape(1, 2))

    r00, r01, r10, r11 = prop_k(edges2d, g2T[0], g2T[1], zeros1)

    outT = pl.pallas_call(
        _dense3_body,
        out_shape=jax.ShapeDtypeStruct((2, npad), jnp.float32),
    )(r00.reshape(1, npad), r01.reshape(1, npad), r10.reshape(1, npad),
      r11.reshape(1, npad), g2T, dis, b2.reshape(1, 2))
    return outT.T[:n, :]


# speedup vs baseline: 134.1404x; 2.2150x over previous
import functools

import jax
import jax.numpy as jnp
from jax import lax
from jax.experimental import pallas as pl
from jax.experimental.pallas import tpu as pltpu
from jax.experimental.pallas import tpu_sc as plsc

NC = 2
NS = 16
NW = NC * NS


GRP = 1024
KCH = 8
NBUF = 4


def _sc_mesh():
    return plsc.VectorSubcoreMesh(core_axis_name="c", subcore_axis_name="s")


_SC_PARAMS = pltpu.CompilerParams(use_tc_tiling_on_sc=False)


def _make_degree_kernel(NP, RPW, GRP=GRP, KCH=KCH):
    RH = RPW * NW
    nchunk = RPW // KCH
    rpt = NP // NS

    @functools.partial(
        pl.kernel,
        out_type=[jax.ShapeDtypeStruct((NP,), jnp.float32),
                  jax.ShapeDtypeStruct((NP,), jnp.float32)],
        mesh=_sc_mesh(),
        compiler_params=_SC_PARAMS,
        scratch_types=[
            pltpu.VMEM((KCH, GRP), jnp.int32),
            pltpu.VMEM((GRP,), jnp.float32),
            pltpu.VMEM((rpt,), jnp.float32),
            pltpu.VMEM_SHARED((NP,), jnp.float32),
            pltpu.SemaphoreType.DMA,
        ],
    )
    def deg_kernel(edge_hbm, zeros_hbm, out0_hbm, out1_hbm,
                   idx_v, ones_v, stage_v, acc_sh, sem):
        cid = lax.axis_index("c")
        sid = lax.axis_index("s")
        sl = pl.ds(sid * rpt, rpt)
        pltpu.sync_copy(zeros_hbm.at[sl], stage_v)
        pltpu.sync_copy(stage_v, acc_sh.at[sl])
        def fill(i, _):
            ones_v[pl.ds(i * 16, 16)] = jnp.ones((16,), jnp.float32)
            return 0
        lax.fori_loop(0, GRP // 16, fill, 0)
        plsc.subcore_barrier()
        base = RH + (cid * NS + sid) * RPW
        def body(i, _):
            pltpu.sync_copy(edge_hbm.at[pl.ds(base + i * KCH, KCH), :], idx_v)
            ss = [pltpu.async_copy(ones_v, acc_sh.at[idx_v.at[j]], sem,
                                   add=True)
                  for j in range(KCH)]
            for s in ss:
                s.wait()
            return 0
        lax.fori_loop(0, nchunk, body, 0)
        plsc.subcore_barrier()
        pltpu.sync_copy(acc_sh.at[sl], stage_v)
        @pl.when(cid == 0)
        def _():
            pltpu.sync_copy(stage_v, out0_hbm.at[sl])
        @pl.when(cid == 1)
        def _():
            pltpu.sync_copy(stage_v, out1_hbm.at[sl])

    return deg_kernel


def _make_prop_kernel(NP, RPW, GRP=GRP, KCH=KCH):
    RH = RPW * NW
    nchunk = RPW // KCH
    rpt = NP // NS

    @functools.partial(
        pl.kernel,
        out_type=[jax.ShapeDtypeStruct((NP,), jnp.float32),
                  jax.ShapeDtypeStruct((NP,), jnp.float32),
                  jax.ShapeDtypeStruct((NP,), jnp.float32),
                  jax.ShapeDtypeStruct((NP,), jnp.float32)],
        mesh=_sc_mesh(),
        compiler_params=_SC_PARAMS,
        scratch_types=[
            pltpu.VMEM((KCH, GRP), jnp.int32),
            pltpu.VMEM((KCH, GRP), jnp.int32),
            pltpu.VMEM((NBUF, GRP), jnp.float32),
            pltpu.VMEM((NBUF, GRP), jnp.float32),
            pltpu.VMEM((rpt,), jnp.float32),
            pltpu.VMEM_SHARED((NP,), jnp.float32),
            pltpu.VMEM_SHARED((NP,), jnp.float32),
            pltpu.VMEM_SHARED((NP,), jnp.float32),
            pltpu.VMEM_SHARED((NP,), jnp.float32),
            pltpu.SemaphoreType.DMA,
            pltpu.SemaphoreType.DMA,
            pltpu.SemaphoreType.DMA,
            pltpu.SemaphoreType.DMA,
        ],
    )
    def prop_kernel(edge_hbm, g0_hbm, g1_hbm, zeros_hbm,
                    o00_hbm, o01_hbm, o10_hbm, o11_hbm,
                    si_v, di_v, r0_v, r1_v, stage_v,
                    tab0_sh, tab1_sh, acc0_sh, acc1_sh,
                    sem_g0, sem_g1, sem_s0, sem_s1):
        cid = lax.axis_index("c")
        sid = lax.axis_index("s")
        sl = pl.ds(sid * rpt, rpt)
        pltpu.sync_copy(zeros_hbm.at[sl], stage_v)
        pltpu.sync_copy(stage_v, acc0_sh.at[sl])
        pltpu.sync_copy(stage_v, acc1_sh.at[sl])
        pltpu.sync_copy(g0_hbm.at[sl], stage_v)
        pltpu.sync_copy(stage_v, tab0_sh.at[sl])
        pltpu.sync_copy(g1_hbm.at[sl], stage_v)
        pltpu.sync_copy(stage_v, tab1_sh.at[sl])
        plsc.subcore_barrier()
        wbase = (cid * NS + sid) * RPW
        def body(i, _):
            r0 = wbase + i * KCH
            pltpu.sync_copy(edge_hbm.at[pl.ds(r0, KCH), :], si_v)
            pltpu.sync_copy(edge_hbm.at[pl.ds(RH + r0, KCH), :], di_v)
            ga = [None] * KCH
            gb = [None] * KCH
            sa = [None] * KCH
            sb = [None] * KCH
            def fire_scatter(j):
                b = j % NBUF
                ga[j].wait()
                gb[j].wait()
                sa[j] = pltpu.async_copy(
                    r0_v.at[b], acc0_sh.at[di_v.at[j]], sem_s0, add=True)
                sb[j] = pltpu.async_copy(
                    r1_v.at[b], acc1_sh.at[di_v.at[j]], sem_s1, add=True)
            for j in range(KCH):
                b = j % NBUF
                if j >= NBUF:
                    sa[j - NBUF].wait()
                    sb[j - NBUF].wait()
                ga[j] = pltpu.async_copy(tab0_sh.at[si_v.at[j]],
                                         r0_v.at[b], sem_g0)
                gb[j] = pltpu.async_copy(tab1_sh.at[si_v.at[j]],
                                         r1_v.at[b], sem_g1)
                if j >= 1:
                    fire_scatter(j - 1)
            fire_scatter(KCH - 1)
            for j in range(max(0, KCH - NBUF), KCH):
                sa[j].wait()
                sb[j].wait()
            return 0
        lax.fori_loop(0, nchunk, body, 0)
        plsc.subcore_barrier()
        @pl.when(cid == 0)
        def _():
            pltpu.sync_copy(acc0_sh.at[sl], stage_v)
            pltpu.sync_copy(stage_v, o00_hbm.at[sl])
            pltpu.sync_copy(acc1_sh.at[sl], stage_v)
            pltpu.sync_copy(stage_v, o01_hbm.at[sl])
        @pl.when(cid == 1)
        def _():
            pltpu.sync_copy(acc0_sh.at[sl], stage_v)
            pltpu.sync_copy(stage_v, o10_hbm.at[sl])
            pltpu.sync_copy(acc1_sh.at[sl], stage_v)
            pltpu.sync_copy(stage_v, o11_hbm.at[sl])

    return prop_kernel


def _dense1_body(d0_ref, d1_ref, xT_ref, dis_ref, g1T_ref):
    deg = d0_ref[...] + d1_ref[...] + 1.0
    dis = lax.rsqrt(deg)
    dis_ref[...] = dis
    g1T_ref[...] = xT_ref[...] * dis


def _dense2_body(q00_ref, q01_ref, q10_ref, q11_ref, g1T_ref, dis_ref,
                 W1_ref, b1_ref, W2_ref, b2_ref, g2T_ref):
    dis = dis_ref[...]
    s0 = dis * (q00_ref[...] + q10_ref[...] + g1T_ref[0:1, :])
    s1 = dis * (q01_ref[...] + q11_ref[...] + g1T_ref[1:2, :])
    hs = []
    for j in range(4):
        hj = (s0 * W1_ref[0:1, j:j + 1] + s1 * W1_ref[1:2, j:j + 1]
              + b1_ref[0:1, j:j + 1])
        hs.append(jnp.maximum(hj, 0.0))
    ts = []
    for f in range(2):
        tf = (hs[0] * W2_ref[0:1, f:f + 1] + hs[1] * W2_ref[1:2, f:f + 1]
              + hs[2] * W2_ref[2:3, f:f + 1] + hs[3] * W2_ref[3:4, f:f + 1])
        ts.append(tf)
    g2T_ref[...] = jnp.concatenate(ts, axis=0) * dis


def _dense3_body(q00_ref, q01_ref, q10_ref, q11_ref, g2T_ref, dis_ref,
                 b2_ref, outT_ref):
    dis = dis_ref[...]
    s0 = dis * (q00_ref[...] + q10_ref[...] + g2T_ref[0:1, :]) + b2_ref[0:1, 0:1]
    s1 = dis * (q01_ref[...] + q11_ref[...] + g2T_ref[1:2, :]) + b2_ref[0:1, 1:2]
    m = jnp.maximum(s0, s1)
    z0, z1 = s0 - m, s1 - m
    lse = jnp.log(jnp.exp(z0) + jnp.exp(z1))
    outT_ref[...] = jnp.concatenate([z0 - lse, z1 - lse], axis=0)


def kernel(x, edge_index, W1, b1, W2, b2):
    n, f_in = x.shape
    e = edge_index.shape[1]
    npad = ((n + 64 + NS * 8 - 1) // (NS * 8)) * (NS * 8)
    rpw = -(-e // (GRP * NW * KCH)) * KCH
    ep = rpw * NW * GRP
    pad_e = ep - e
    pad_idx = npad - 64 + (jnp.arange(pad_e, dtype=edge_index.dtype) % 64)
    srcp = jnp.concatenate([edge_index[0], pad_idx])
    dstp = jnp.concatenate([edge_index[1], pad_idx])
    edges2d = jnp.concatenate([srcp, dstp]).reshape(-1, GRP)

    xTp = jnp.concatenate(
        [x, jnp.zeros((npad - n, f_in), jnp.float32)], axis=0).T
    zeros1 = jnp.zeros((npad,), jnp.float32)

    deg_k = _make_degree_kernel(npad, rpw)
    prop_k = _make_prop_kernel(npad, rpw)

    d0, d1 = deg_k(edges2d, zeros1)

    dis, g1T = pl.pallas_call(
        _dense1_body,
        out_shape=[
            jax.ShapeDtypeStruct((1, npad), jnp.float32),
            jax.ShapeDtypeStruct((2, npad), jnp.float32),
        ],
    )(d0.reshape(1, npad), d1.reshape(1, npad), xTp)

    q00, q01, q10, q11 = prop_k(edges2d, g1T[0], g1T[1], zeros1)

    g2T = pl.pallas_call(
        _dense2_body,
        out_shape=jax.ShapeDtypeStruct((2, npad), jnp.float32),
    )(q00.reshape(1, npad), q01.reshape(1, npad), q10.reshape(1, npad),
      q11.reshape(1, npad), g1T, dis, W1, b1.reshape(1, 4), W2,
      b2.reshape(1, 2))

    r00, r01, r10, r11 = prop_k(edges2d, g2T[0], g2T[1], zeros1)

    outT = pl.pallas_call(
        _dense3_body,
        out_shape=jax.ShapeDtypeStruct((2, npad), jnp.float32),
    )(r00.reshape(1, npad), r01.reshape(1, npad), r10.reshape(1, npad),
      r11.reshape(1, npad), g2T, dis, b2.reshape(1, 2))
    return outT.T[:n, :]


# speedup vs baseline: 134.9965x; 1.0064x over previous
import functools

import jax
import jax.numpy as jnp
from jax import lax
from jax.experimental import pallas as pl
from jax.experimental.pallas import tpu as pltpu
from jax.experimental.pallas import tpu_sc as plsc

NC = 2
NS = 16
NW = NC * NS


GRP = 1024
KCH = 8
NBUF = 4


def _sc_mesh():
    return plsc.VectorSubcoreMesh(core_axis_name="c", subcore_axis_name="s")


_SC_PARAMS = pltpu.CompilerParams(use_tc_tiling_on_sc=False)


def _make_degree_kernel(NP, RPW, GRP=GRP, KCH=KCH):
    RH = RPW * NW
    nchunk = RPW // KCH
    rpt = NP // NS

    @functools.partial(
        pl.kernel,
        out_type=[jax.ShapeDtypeStruct((NP,), jnp.float32),
                  jax.ShapeDtypeStruct((NP,), jnp.float32)],
        mesh=_sc_mesh(),
        compiler_params=_SC_PARAMS,
        scratch_types=[
            pltpu.VMEM((KCH, GRP), jnp.int32),
            pltpu.VMEM((GRP,), jnp.float32),
            pltpu.VMEM((rpt,), jnp.float32),
            pltpu.VMEM_SHARED((NP,), jnp.float32),
            pltpu.SemaphoreType.DMA,
        ],
    )
    def deg_kernel(edge_hbm, zeros_hbm, out0_hbm, out1_hbm,
                   idx_v, ones_v, stage_v, acc_sh, sem):
        cid = lax.axis_index("c")
        sid = lax.axis_index("s")
        sl = pl.ds(sid * rpt, rpt)
        pltpu.sync_copy(zeros_hbm.at[sl], stage_v)
        pltpu.sync_copy(stage_v, acc_sh.at[sl])
        def fill(i, _):
            ones_v[pl.ds(i * 16, 16)] = jnp.ones((16,), jnp.float32)
            return 0
        lax.fori_loop(0, GRP // 16, fill, 0)
        plsc.subcore_barrier()
        base = RH + (cid * NS + sid) * RPW
        def body(i, _):
            pltpu.sync_copy(edge_hbm.at[pl.ds(base + i * KCH, KCH), :], idx_v)
            ss = [pltpu.async_copy(ones_v, acc_sh.at[idx_v.at[j]], sem,
                                   add=True)
                  for j in range(KCH)]
            for s in ss:
                s.wait()
            return 0
        lax.fori_loop(0, nchunk, body, 0)
        plsc.subcore_barrier()
        pltpu.sync_copy(acc_sh.at[sl], stage_v)
        @pl.when(cid == 0)
        def _():
            pltpu.sync_copy(stage_v, out0_hbm.at[sl])
        @pl.when(cid == 1)
        def _():
            pltpu.sync_copy(stage_v, out1_hbm.at[sl])

    return deg_kernel


def _make_prop_kernel(NP, RPW, GRP=GRP, KCH=KCH):
    RH = RPW * NW
    nchunk = RPW // KCH
    rpt = NP // NS

    @functools.partial(
        pl.kernel,
        out_type=[jax.ShapeDtypeStruct((NP,), jnp.float32),
                  jax.ShapeDtypeStruct((NP,), jnp.float32),
                  jax.ShapeDtypeStruct((NP,), jnp.float32),
                  jax.ShapeDtypeStruct((NP,), jnp.float32)],
        mesh=_sc_mesh(),
        compiler_params=_SC_PARAMS,
        scratch_types=[
            pltpu.VMEM((KCH, GRP), jnp.int32),
            pltpu.VMEM((KCH, GRP), jnp.int32),
            pltpu.VMEM((KCH, GRP), jnp.int32),
            pltpu.VMEM((KCH, GRP), jnp.int32),
            pltpu.VMEM((NBUF, GRP), jnp.float32),
            pltpu.VMEM((NBUF, GRP), jnp.float32),
            pltpu.VMEM((rpt,), jnp.float32),
            pltpu.VMEM_SHARED((NP,), jnp.float32),
            pltpu.VMEM_SHARED((NP,), jnp.float32),
            pltpu.VMEM_SHARED((NP,), jnp.float32),
            pltpu.VMEM_SHARED((NP,), jnp.float32),
            pltpu.SemaphoreType.DMA,
            pltpu.SemaphoreType.DMA,
            pltpu.SemaphoreType.DMA,
            pltpu.SemaphoreType.DMA,
            pltpu.SemaphoreType.DMA,
        ],
    )
    def prop_kernel(edge_hbm, g0_hbm, g1_hbm, zeros_hbm,
                    o00_hbm, o01_hbm, o10_hbm, o11_hbm,
                    si0_v, di0_v, si1_v, di1_v, r0_v, r1_v, stage_v,
                    tab0_sh, tab1_sh, acc0_sh, acc1_sh,
                    sem_g0, sem_g1, sem_s0, sem_s1, sem_i):
        cid = lax.axis_index("c")
        sid = lax.axis_index("s")
        sl = pl.ds(sid * rpt, rpt)
        pltpu.sync_copy(zeros_hbm.at[sl], stage_v)
        pltpu.sync_copy(stage_v, acc0_sh.at[sl])
        pltpu.sync_copy(stage_v, acc1_sh.at[sl])
        pltpu.sync_copy(g0_hbm.at[sl], stage_v)
        pltpu.sync_copy(stage_v, tab0_sh.at[sl])
        pltpu.sync_copy(g1_hbm.at[sl], stage_v)
        pltpu.sync_copy(stage_v, tab1_sh.at[sl])
        plsc.subcore_barrier()
        wbase = (cid * NS + sid) * RPW

        def fire_idx_load(c, si_b, di_b):
            r0 = wbase + c * KCH
            pltpu.async_copy(edge_hbm.at[pl.ds(r0, KCH), :], si_b, sem_i)
            pltpu.async_copy(edge_hbm.at[pl.ds(RH + r0, KCH), :], di_b, sem_i)

        def wait_idx_load(si_b, di_b):
            pltpu.make_async_copy(edge_hbm.at[pl.ds(0, KCH), :], si_b,
                                  sem_i).wait()
            pltpu.make_async_copy(edge_hbm.at[pl.ds(0, KCH), :], di_b,
                                  sem_i).wait()

        def process_chunk(si_b, di_b):
            ga = [None] * KCH
            gb = [None] * KCH
            sa = [None] * KCH
            sb = [None] * KCH
            def fire_scatter(j):
                b = j % NBUF
                ga[j].wait()
                gb[j].wait()
                sa[j] = pltpu.async_copy(
                    r0_v.at[b], acc0_sh.at[di_b.at[j]], sem_s0, add=True)
                sb[j] = pltpu.async_copy(
                    r1_v.at[b], acc1_sh.at[di_b.at[j]], sem_s1, add=True)
            for j in range(KCH):
                b = j % NBUF
                if j >= NBUF:
                    sa[j - NBUF].wait()
                    sb[j - NBUF].wait()
                ga[j] = pltpu.async_copy(tab0_sh.at[si_b.at[j]],
                                         r0_v.at[b], sem_g0)
                gb[j] = pltpu.async_copy(tab1_sh.at[si_b.at[j]],
                                         r1_v.at[b], sem_g1)
                if j >= 1:
                    fire_scatter(j - 1)
            fire_scatter(KCH - 1)
            for j in range(max(0, KCH - NBUF), KCH):
                sa[j].wait()
                sb[j].wait()

        fire_idx_load(0, si0_v, di0_v)
        def body(i2, _):
            c0 = 2 * i2
            wait_idx_load(si0_v, di0_v)
            fire_idx_load(c0 + 1, si1_v, di1_v)
            process_chunk(si0_v, di0_v)
            wait_idx_load(si1_v, di1_v)
            @pl.when(i2 + 1 < nchunk // 2)
            def _():
                fire_idx_load(c0 + 2, si0_v, di0_v)
            process_chunk(si1_v, di1_v)
            return 0
        lax.fori_loop(0, nchunk // 2, body, 0)
        plsc.subcore_barrier()
        @pl.when(cid == 0)
        def _():
            pltpu.sync_copy(acc0_sh.at[sl], stage_v)
            pltpu.sync_copy(stage_v, o00_hbm.at[sl])
            pltpu.sync_copy(acc1_sh.at[sl], stage_v)
            pltpu.sync_copy(stage_v, o01_hbm.at[sl])
        @pl.when(cid == 1)
        def _():
            pltpu.sync_copy(acc0_sh.at[sl], stage_v)
            pltpu.sync_copy(stage_v, o10_hbm.at[sl])
            pltpu.sync_copy(acc1_sh.at[sl], stage_v)
            pltpu.sync_copy(stage_v, o11_hbm.at[sl])

    return prop_kernel


def _dense1_body(d0_ref, d1_ref, xT_ref, dis_ref, g1T_ref):
    deg = d0_ref[...] + d1_ref[...] + 1.0
    dis = lax.rsqrt(deg)
    dis_ref[...] = dis
    g1T_ref[...] = xT_ref[...] * dis


def _dense2_body(q00_ref, q01_ref, q10_ref, q11_ref, g1T_ref, dis_ref,
                 W1_ref, b1_ref, W2_ref, b2_ref, g2T_ref):
    dis = dis_ref[...]
    s0 = dis * (q00_ref[...] + q10_ref[...] + g1T_ref[0:1, :])
    s1 = dis * (q01_ref[...] + q11_ref[...] + g1T_ref[1:2, :])
    hs = []
    for j in range(4):
        hj = (s0 * W1_ref[0:1, j:j + 1] + s1 * W1_ref[1:2, j:j + 1]
              + b1_ref[0:1, j:j + 1])
        hs.append(jnp.maximum(hj, 0.0))
    ts = []
    for f in range(2):
        tf = (hs[0] * W2_ref[0:1, f:f + 1] + hs[1] * W2_ref[1:2, f:f + 1]
              + hs[2] * W2_ref[2:3, f:f + 1] + hs[3] * W2_ref[3:4, f:f + 1])
        ts.append(tf)
    g2T_ref[...] = jnp.concatenate(ts, axis=0) * dis


def _dense3_body(q00_ref, q01_ref, q10_ref, q11_ref, g2T_ref, dis_ref,
                 b2_ref, outT_ref):
    dis = dis_ref[...]
    s0 = dis * (q00_ref[...] + q10_ref[...] + g2T_ref[0:1, :]) + b2_ref[0:1, 0:1]
    s1 = dis * (q01_ref[...] + q11_ref[...] + g2T_ref[1:2, :]) + b2_ref[0:1, 1:2]
    m = jnp.maximum(s0, s1)
    z0, z1 = s0 - m, s1 - m
    lse = jnp.log(jnp.exp(z0) + jnp.exp(z1))
    outT_ref[...] = jnp.concatenate([z0 - lse, z1 - lse], axis=0)


def kernel(x, edge_index, W1, b1, W2, b2):
    n, f_in = x.shape
    e = edge_index.shape[1]
    npad = ((n + 64 + NS * 8 - 1) // (NS * 8)) * (NS * 8)
    rpw = -(-e // (GRP * NW * KCH * 2)) * (KCH * 2)
    ep = rpw * NW * GRP
    pad_e = ep - e
    pad_idx = npad - 64 + (jnp.arange(pad_e, dtype=edge_index.dtype) % 64)
    srcp = jnp.concatenate([edge_index[0], pad_idx])
    dstp = jnp.concatenate([edge_index[1], pad_idx])
    edges2d = jnp.concatenate([srcp, dstp]).reshape(-1, GRP)

    xTp = jnp.concatenate(
        [x, jnp.zeros((npad - n, f_in), jnp.float32)], axis=0).T
    zeros1 = jnp.zeros((npad,), jnp.float32)

    deg_k = _make_degree_kernel(npad, rpw)
    prop_k = _make_prop_kernel(npad, rpw)

    d0, d1 = deg_k(edges2d, zeros1)

    dis, g1T = pl.pallas_call(
        _dense1_body,
        out_shape=[
            jax.ShapeDtypeStruct((1, npad), jnp.float32),
            jax.ShapeDtypeStruct((2, npad), jnp.float32),
        ],
    )(d0.reshape(1, npad), d1.reshape(1, npad), xTp)

    q00, q01, q10, q11 = prop_k(edges2d, g1T[0], g1T[1], zeros1)

    g2T = pl.pallas_call(
        _dense2_body,
        out_shape=jax.ShapeDtypeStruct((2, npad), jnp.float32),
    )(q00.reshape(1, npad), q01.reshape(1, npad), q10.reshape(1, npad),
      q11.reshape(1, npad), g1T, dis, W1, b1.reshape(1, 4), W2,
      b2.reshape(1, 2))

    r00, r01, r10, r11 = prop_k(edges2d, g2T[0], g2T[1], zeros1)

    outT = pl.pallas_call(
        _dense3_body,
        out_shape=jax.ShapeDtypeStruct((2, npad), jnp.float32),
    )(r00.reshape(1, npad), r01.reshape(1, npad), r10.reshape(1, npad),
      r11.reshape(1, npad), g2T, dis, b2.reshape(1, 2))
    return outT.T[:n, :]


# speedup vs baseline: 143.5161x; 1.0631x over previous
import functools

import jax
import jax.numpy as jnp
from jax import lax
from jax.experimental import pallas as pl
from jax.experimental.pallas import tpu as pltpu
from jax.experimental.pallas import tpu_sc as plsc

NC = 2
NS = 16
NW = NC * NS


GRP = 1024
KCH = 8
NBUF = 4


def _sc_mesh():
    return plsc.VectorSubcoreMesh(core_axis_name="c", subcore_axis_name="s")


_SC_PARAMS = pltpu.CompilerParams(use_tc_tiling_on_sc=False)


def _make_degree_kernel(NP, RPW, GRP=GRP, KCH=KCH):
    RH = RPW * NW
    nchunk = RPW // KCH
    rpt = NP // NS

    @functools.partial(
        pl.kernel,
        out_type=[jax.ShapeDtypeStruct((NP,), jnp.float32),
                  jax.ShapeDtypeStruct((NP,), jnp.float32)],
        mesh=_sc_mesh(),
        compiler_params=_SC_PARAMS,
        scratch_types=[
            pltpu.VMEM((KCH, GRP), jnp.int32),
            pltpu.VMEM((GRP,), jnp.float32),
            pltpu.VMEM((rpt,), jnp.float32),
            pltpu.VMEM_SHARED((NP,), jnp.float32),
            pltpu.SemaphoreType.DMA,
        ],
    )
    def deg_kernel(edge_hbm, zeros_hbm, out0_hbm, out1_hbm,
                   idx_v, ones_v, stage_v, acc_sh, sem):
        cid = lax.axis_index("c")
        sid = lax.axis_index("s")
        sl = pl.ds(sid * rpt, rpt)
        pltpu.sync_copy(zeros_hbm.at[sl], stage_v)
        pltpu.sync_copy(stage_v, acc_sh.at[sl])
        def fill(i, _):
            ones_v[pl.ds(i * 16, 16)] = jnp.ones((16,), jnp.float32)
            return 0
        lax.fori_loop(0, GRP // 16, fill, 0)
        plsc.subcore_barrier()
        base = RH + (cid * NS + sid) * RPW
        def body(i, _):
            pltpu.sync_copy(edge_hbm.at[pl.ds(base + i * KCH, KCH), :], idx_v)
            ss = [pltpu.async_copy(ones_v, acc_sh.at[idx_v.at[j]], sem,
                                   add=True)
                  for j in range(KCH)]
            for s in ss:
                s.wait()
            return 0
        lax.fori_loop(0, nchunk, body, 0)
        plsc.subcore_barrier()
        pltpu.sync_copy(acc_sh.at[sl], stage_v)
        @pl.when(cid == 0)
        def _():
            pltpu.sync_copy(stage_v, out0_hbm.at[sl])
        @pl.when(cid == 1)
        def _():
            pltpu.sync_copy(stage_v, out1_hbm.at[sl])

    return deg_kernel


def _make_prop_kernel(NP, RPW, GRP=GRP, KCH=KCH):
    RH = RPW * NW
    nchunk = RPW // KCH
    rpt = NP // NS

    @functools.partial(
        pl.kernel,
        out_type=[jax.ShapeDtypeStruct((NP,), jnp.float32),
                  jax.ShapeDtypeStruct((NP,), jnp.float32),
                  jax.ShapeDtypeStruct((NP,), jnp.float32),
                  jax.ShapeDtypeStruct((NP,), jnp.float32)],
        mesh=_sc_mesh(),
        compiler_params=_SC_PARAMS,
        scratch_types=[
            pltpu.VMEM((KCH, GRP), jnp.int32),
            pltpu.VMEM((KCH, GRP), jnp.int32),
            pltpu.VMEM((KCH, GRP), jnp.int32),
            pltpu.VMEM((KCH, GRP), jnp.int32),
            pltpu.VMEM((NBUF, GRP), jnp.float32),
            pltpu.VMEM((NBUF, GRP), jnp.float32),
            pltpu.VMEM((rpt,), jnp.float32),
            pltpu.VMEM_SHARED((NP,), jnp.float32),
            pltpu.VMEM_SHARED((NP,), jnp.float32),
            pltpu.VMEM_SHARED((NP,), jnp.float32),
            pltpu.VMEM_SHARED((NP,), jnp.float32),
            pltpu.SemaphoreType.DMA,
            pltpu.SemaphoreType.DMA,
            pltpu.SemaphoreType.DMA,
            pltpu.SemaphoreType.DMA,
            pltpu.SemaphoreType.DMA,
        ],
    )
    def prop_kernel(edge_hbm, g0_hbm, g1_hbm, zeros_hbm,
                    o00_hbm, o01_hbm, o10_hbm, o11_hbm,
                    si0_v, di0_v, si1_v, di1_v, r0_v, r1_v, stage_v,
                    tab0_sh, tab1_sh, acc0_sh, acc1_sh,
                    sem_g0, sem_g1, sem_s0, sem_s1, sem_i):
        cid = lax.axis_index("c")
        sid = lax.axis_index("s")
        sl = pl.ds(sid * rpt, rpt)
        pltpu.sync_copy(zeros_hbm.at[sl], stage_v)
        pltpu.sync_copy(stage_v, acc0_sh.at[sl])
        pltpu.sync_copy(stage_v, acc1_sh.at[sl])
        pltpu.sync_copy(g0_hbm.at[sl], stage_v)
        pltpu.sync_copy(stage_v, tab0_sh.at[sl])
        pltpu.sync_copy(g1_hbm.at[sl], stage_v)
        pltpu.sync_copy(stage_v, tab1_sh.at[sl])
        plsc.subcore_barrier()
        wbase = (cid * NS + sid) * RPW

        def fire_idx_load(c, si_b, di_b):
            r0 = wbase + c * KCH
            pltpu.async_copy(edge_hbm.at[pl.ds(r0, KCH), :], si_b, sem_i)
            pltpu.async_copy(edge_hbm.at[pl.ds(RH + r0, KCH), :], di_b, sem_i)

        def wait_idx_load(si_b, di_b):
            pltpu.make_async_copy(edge_hbm.at[pl.ds(0, KCH), :], si_b,
                                  sem_i).wait()
            pltpu.make_async_copy(edge_hbm.at[pl.ds(0, KCH), :], di_b,
                                  sem_i).wait()

        def process_chunk(si_b, di_b):
            ga = [None] * KCH
            gb = [None] * KCH
            sa = [None] * KCH
            sb = [None] * KCH
            def fire_scatter(j):
                b = j % NBUF
                ga[j].wait()
                gb[j].wait()
                sa[j] = pltpu.async_copy(
                    r0_v.at[b], acc0_sh.at[di_b.at[j]], sem_s0, add=True)
                sb[j] = pltpu.async_copy(
                    r1_v.at[b], acc1_sh.at[di_b.at[j]], sem_s1, add=True)
            for j in range(KCH):
                b = j % NBUF
                if j >= NBUF:
                    sa[j - NBUF].wait()
                    sb[j - NBUF].wait()
                ga[j] = pltpu.async_copy(tab0_sh.at[si_b.at[j]],
                                         r0_v.at[b], sem_g0)
                gb[j] = pltpu.async_copy(tab1_sh.at[si_b.at[j]],
                                         r1_v.at[b], sem_g1)
                if j >= 1:
                    fire_scatter(j - 1)
            fire_scatter(KCH - 1)
            for j in range(max(0, KCH - NBUF), KCH):
                sa[j].wait()
                sb[j].wait()

        fire_idx_load(0, si0_v, di0_v)
        def body(i2, _):
            c0 = 2 * i2
            wait_idx_load(si0_v, di0_v)
            fire_idx_load(c0 + 1, si1_v, di1_v)
            process_chunk(si0_v, di0_v)
            wait_idx_load(si1_v, di1_v)
            @pl.when(c0 + 2 < nchunk)
            def _():
                fire_idx_load(c0 + 2, si0_v, di0_v)
            process_chunk(si1_v, di1_v)
            return 0
        lax.fori_loop(0, nchunk // 2, body, 0)
        if nchunk % 2 == 1:
            wait_idx_load(si0_v, di0_v)
            process_chunk(si0_v, di0_v)
        plsc.subcore_barrier()
        @pl.when(cid == 0)
        def _():
            pltpu.sync_copy(acc0_sh.at[sl], stage_v)
            pltpu.sync_copy(stage_v, o00_hbm.at[sl])
            pltpu.sync_copy(acc1_sh.at[sl], stage_v)
            pltpu.sync_copy(stage_v, o01_hbm.at[sl])
        @pl.when(cid == 1)
        def _():
            pltpu.sync_copy(acc0_sh.at[sl], stage_v)
            pltpu.sync_copy(stage_v, o10_hbm.at[sl])
            pltpu.sync_copy(acc1_sh.at[sl], stage_v)
            pltpu.sync_copy(stage_v, o11_hbm.at[sl])

    return prop_kernel


def _dense1_body(d0_ref, d1_ref, xT_ref, dis_ref, g1T_ref):
    deg = d0_ref[...] + d1_ref[...] + 1.0
    dis = lax.rsqrt(deg)
    dis_ref[...] = dis
    g1T_ref[...] = xT_ref[...] * dis


def _dense2_body(q00_ref, q01_ref, q10_ref, q11_ref, g1T_ref, dis_ref,
                 W1_ref, b1_ref, W2_ref, b2_ref, g2T_ref):
    dis = dis_ref[...]
    s0 = dis * (q00_ref[...] + q10_ref[...] + g1T_ref[0:1, :])
    s1 = dis * (q01_ref[...] + q11_ref[...] + g1T_ref[1:2, :])
    hs = []
    for j in range(4):
        hj = (s0 * W1_ref[0:1, j:j + 1] + s1 * W1_ref[1:2, j:j + 1]
              + b1_ref[0:1, j:j + 1])
        hs.append(jnp.maximum(hj, 0.0))
    ts = []
    for f in range(2):
        tf = (hs[0] * W2_ref[0:1, f:f + 1] + hs[1] * W2_ref[1:2, f:f + 1]
              + hs[2] * W2_ref[2:3, f:f + 1] + hs[3] * W2_ref[3:4, f:f + 1])
        ts.append(tf)
    g2T_ref[...] = jnp.concatenate(ts, axis=0) * dis


def _dense3_body(q00_ref, q01_ref, q10_ref, q11_ref, g2T_ref, dis_ref,
                 b2_ref, outT_ref):
    dis = dis_ref[...]
    s0 = dis * (q00_ref[...] + q10_ref[...] + g2T_ref[0:1, :]) + b2_ref[0:1, 0:1]
    s1 = dis * (q01_ref[...] + q11_ref[...] + g2T_ref[1:2, :]) + b2_ref[0:1, 1:2]
    m = jnp.maximum(s0, s1)
    z0, z1 = s0 - m, s1 - m
    lse = jnp.log(jnp.exp(z0) + jnp.exp(z1))
    outT_ref[...] = jnp.concatenate([z0 - lse, z1 - lse], axis=0)


def kernel(x, edge_index, W1, b1, W2, b2):
    n, f_in = x.shape
    e = edge_index.shape[1]
    npad = ((n + 64 + NS * 8 - 1) // (NS * 8)) * (NS * 8)
    rpw = -(-e // (GRP * NW * KCH)) * KCH
    ep = rpw * NW * GRP
    pad_e = ep - e
    pad_idx = npad - 64 + (jnp.arange(pad_e, dtype=edge_index.dtype) % 64)
    edges2d = jnp.concatenate(
        [edge_index[0], pad_idx, edge_index[1], pad_idx]).reshape(-1, GRP)

    xTp = jnp.concatenate(
        [x, jnp.zeros((npad - n, f_in), jnp.float32)], axis=0).T
    zeros1 = jnp.zeros((npad,), jnp.float32)

    deg_k = _make_degree_kernel(npad, rpw)
    prop_k = _make_prop_kernel(npad, rpw)

    d0, d1 = deg_k(edges2d, zeros1)

    dis, g1T = pl.pallas_call(
        _dense1_body,
        out_shape=[
            jax.ShapeDtypeStruct((1, npad), jnp.float32),
            jax.ShapeDtypeStruct((2, npad), jnp.float32),
        ],
    )(d0.reshape(1, npad), d1.reshape(1, npad), xTp)

    q00, q01, q10, q11 = prop_k(edges2d, g1T[0], g1T[1], zeros1)

    g2T = pl.pallas_call(
        _dense2_body,
        out_shape=jax.ShapeDtypeStruct((2, npad), jnp.float32),
    )(q00.reshape(1, npad), q01.reshape(1, npad), q10.reshape(1, npad),
      q11.reshape(1, npad), g1T, dis, W1, b1.reshape(1, 4), W2,
      b2.reshape(1, 2))

    r00, r01, r10, r11 = prop_k(edges2d, g2T[0], g2T[1], zeros1)

    outT = pl.pallas_call(
        _dense3_body,
        out_shape=jax.ShapeDtypeStruct((2, npad), jnp.float32),
    )(r00.reshape(1, npad), r01.reshape(1, npad), r10.reshape(1, npad),
      r11.reshape(1, npad), g2T, dis, b2.reshape(1, 2))
    return outT.T[:n, :]
